# Initial kernel scaffold; baseline (speedup 1.0000x reference)
import functools

import jax
import jax.numpy as jnp
from jax import lax
from jax.experimental import pallas as pl
from jax.experimental.pallas import tpu as pltpu
from jax.experimental.pallas import tpu_sc as plsc

N_NODES = 10000
N_PAD = 10240
N_EDGES = 320000
NCORES = 2
NSUB = 16
NW = NCORES * NSUB
BATCH = 128
GROUPS = 79
E_PAD = NW * GROUPS * BATCH
ROWS_PER_TILE = N_PAD // NSUB
PAD_SPREAD = 240


def _sc_prop(F):
    mesh = plsc.VectorSubcoreMesh(core_axis_name="c", subcore_axis_name="s")

    @functools.partial(
        pl.kernel,
        out_type=jax.ShapeDtypeStruct((NCORES, N_PAD, F), jnp.float32),
        mesh=mesh,
        scratch_types=[
            pltpu.VMEM_SHARED((N_PAD, F), jnp.float32),
            pltpu.VMEM((GROUPS, BATCH), jnp.int32),
            pltpu.VMEM((GROUPS, BATCH), jnp.int32),
            pltpu.VMEM((BATCH, F), jnp.float32),
            pltpu.SemaphoreType.DMA,
        ],
    )
    def prop(z_hbm, zeros_hbm, src_hbm, dst_hbm, out_hbm,
             acc, src_v, dst_v, rows, sem):
        c = lax.axis_index("c")
        s = lax.axis_index("s")
        w = c * NSUB + s
        pltpu.sync_copy(src_hbm.at[w], src_v)
        pltpu.sync_copy(dst_hbm.at[w], dst_v)
        r0 = s * ROWS_PER_TILE

        @pl.when(c == 0)
        def _():
            pltpu.sync_copy(z_hbm.at[pl.ds(r0, ROWS_PER_TILE)],
                            acc.at[pl.ds(r0, ROWS_PER_TILE)])

        @pl.when(c != 0)
        def _():
            pltpu.sync_copy(zeros_hbm.at[pl.ds(r0, ROWS_PER_TILE)],
                            acc.at[pl.ds(r0, ROWS_PER_TILE)])

        plsc.subcore_barrier()

        def body(g, carry):
            pltpu.async_copy(z_hbm.at[src_v.at[g]], rows, sem).wait()
            pltpu.sync_copy(rows, acc.at[dst_v.at[g]], add=True)
            return carry

        lax.fori_loop(0, GROUPS, body, 0)
        plsc.subcore_barrier()
        pltpu.sync_copy(acc.at[pl.ds(r0, ROWS_PER_TILE)],
                        out_hbm.at[c, pl.ds(r0, ROWS_PER_TILE)])

    return prop


def _sc_deg():
    mesh = plsc.VectorSubcoreMesh(core_axis_name="c", subcore_axis_name="s")

    @functools.partial(
        pl.kernel,
        out_type=jax.ShapeDtypeStruct((NCORES, N_PAD, 16), jnp.float32),
        mesh=mesh,
        scratch_types=[
            pltpu.VMEM_SHARED((N_PAD, 16), jnp.float32),
            pltpu.VMEM((GROUPS, BATCH), jnp.int32),
            pltpu.VMEM((BATCH, 16), jnp.float32),
        ],
    )
    def deg(dst_hbm, ones_hbm, zeros_hbm, out_hbm, acc, dst_v, ones_v):
        c = lax.axis_index("c")
        s = lax.axis_index("s")
        w = c * NSUB + s
        pltpu.sync_copy(dst_hbm.at[w], dst_v)
        pltpu.sync_copy(ones_hbm, ones_v)
        r0 = s * ROWS_PER_TILE
        pltpu.sync_copy(zeros_hbm.at[pl.ds(r0, ROWS_PER_TILE)],
                        acc.at[pl.ds(r0, ROWS_PER_TILE)])
        plsc.subcore_barrier()

        def body(g, carry):
            pltpu.sync_copy(ones_v, acc.at[dst_v.at[g]], add=True)
            return carry

        lax.fori_loop(0, GROUPS, body, 0)
        plsc.subcore_barrier()
        pltpu.sync_copy(acc.at[pl.ds(r0, ROWS_PER_TILE)],
                        out_hbm.at[c, pl.ds(r0, ROWS_PER_TILE)])

    return deg


_BM = 1024


def _dinv_from_deg(degp):
    def body(p_ref, o_ref):
        deg = p_ref[0, :, 0:1] + p_ref[1, :, 0:1] + 1.0
        safe = jnp.maximum(deg, 1e-12)
        o_ref[...] = jnp.where(deg > 0, lax.rsqrt(safe), 0.0)

    return pl.pallas_call(
        body,
        grid=(N_PAD // _BM,),
        in_specs=[pl.BlockSpec((2, _BM, 16), lambda i: (0, i, 0))],
        out_specs=pl.BlockSpec((_BM, 1), lambda i: (i, 0)),
        out_shape=jax.ShapeDtypeStruct((N_PAD, 1), jnp.float32),
    )(degp)


def _dense_in(h, W, b0, dinv, d0):
    K = h.shape[1]
    dtot = W.shape[1]
    da = dtot - d0

    def body(h_ref, w_ref, b_ref, dv_ref, z0_ref, a_ref):
        prod = jnp.dot(h_ref[...], w_ref[...],
                       preferred_element_type=jnp.float32)
        z0_ref[...] = prod[:, :d0] + b_ref[...]
        a_ref[...] = prod[:, d0:] * dv_ref[...]

    return pl.pallas_call(
        body,
        grid=(N_PAD // _BM,),
        in_specs=[
            pl.BlockSpec((_BM, K), lambda i: (i, 0)),
            pl.BlockSpec((K, dtot), lambda i: (0, 0)),
            pl.BlockSpec((1, d0), lambda i: (0, 0)),
            pl.BlockSpec((_BM, 1), lambda i: (i, 0)),
        ],
        out_specs=[
            pl.BlockSpec((_BM, d0), lambda i: (i, 0)),
            pl.BlockSpec((_BM, da), lambda i: (i, 0)),
        ],
        out_shape=[
            jax.ShapeDtypeStruct((N_PAD, d0), jnp.float32),
            jax.ShapeDtypeStruct((N_PAD, da), jnp.float32),
        ],
    )(h, W, b0, dinv)


def _combine_mid(p, dinv, b1, dh):
    F = p.shape[2]
    da = F - dh

    def body(p_ref, dv_ref, b_ref, o1_ref, g2_ref):
        sm = p_ref[0] + p_ref[1]
        dv = dv_ref[...]
        o1_ref[...] = sm[:, :dh] * dv + b_ref[...]
        g2_ref[...] = sm[:, dh:] * (dv * dv)

    return pl.pallas_call(
        body,
        grid=(N_PAD // _BM,),
        in_specs=[
            pl.BlockSpec((2, _BM, F), lambda i: (0, i, 0)),
            pl.BlockSpec((_BM, 1), lambda i: (i, 0)),
            pl.BlockSpec((1, dh), lambda i: (0, 0)),
        ],
        out_specs=[
            pl.BlockSpec((_BM, dh), lambda i: (i, 0)),
            pl.BlockSpec((_BM, da), lambda i: (i, 0)),
        ],
        out_shape=[
            jax.ShapeDtypeStruct((N_PAD, dh), jnp.float32),
            jax.ShapeDtypeStruct((N_PAD, da), jnp.float32),
        ],
    )(p, dinv, b1)


def _combine_out(z0, out1, q, dinv, b2):
    d0 = z0.shape[1]
    d1 = out1.shape[1]
    d2 = q.shape[2]

    def body(z0_ref, o1_ref, q_ref, dv_ref, b_ref, h_ref):
        o2 = (q_ref[0] + q_ref[1]) * dv_ref[...] + b_ref[...]
        h_ref[...] = jnp.concatenate([z0_ref[...], o1_ref[...], o2], axis=1)

    return pl.pallas_call(
        body,
        grid=(N_PAD // _BM,),
        in_specs=[
            pl.BlockSpec((_BM, d0), lambda i: (i, 0)),
            pl.BlockSpec((_BM, d1), lambda i: (i, 0)),
            pl.BlockSpec((2, _BM, d2), lambda i: (0, i, 0)),
            pl.BlockSpec((_BM, 1), lambda i: (i, 0)),
            pl.BlockSpec((1, d2), lambda i: (0, 0)),
        ],
        out_specs=pl.BlockSpec((_BM, d0 + d1 + d2), lambda i: (i, 0)),
        out_shape=jax.ShapeDtypeStruct((N_PAD, d0 + d1 + d2), jnp.float32),
    )(z0, out1, q, dinv, b2)


def _final_out(z0, q, dinv, b1):
    d0 = z0.shape[1]
    d1 = q.shape[2]

    def body(z0_ref, q_ref, dv_ref, b_ref, h_ref):
        o1 = (q_ref[0] + q_ref[1]) * dv_ref[...] + b_ref[...]
        h_ref[...] = jnp.concatenate([z0_ref[...], o1], axis=1)

    return pl.pallas_call(
        body,
        grid=(N_PAD // _BM,),
        in_specs=[
            pl.BlockSpec((_BM, d0), lambda i: (i, 0)),
            pl.BlockSpec((2, _BM, d1), lambda i: (0, i, 0)),
            pl.BlockSpec((_BM, 1), lambda i: (i, 0)),
            pl.BlockSpec((1, d1), lambda i: (0, 0)),
        ],
        out_specs=pl.BlockSpec((_BM, d0 + d1), lambda i: (i, 0)),
        out_shape=jax.ShapeDtypeStruct((N_PAD, d0 + d1), jnp.float32),
    )(z0, q, dinv, b1)


def kernel(x, edge_index, conv1_W, conv1_b, block_W, block_b, conv3_W, conv3_b):
    f32 = jnp.float32

    xp = jnp.pad(x, ((0, N_PAD - N_NODES), (0, 0)))
    npad = E_PAD - N_EDGES
    pad_ids = (jnp.arange(npad, dtype=jnp.int32) % PAD_SPREAD) + N_NODES
    srcp = jnp.concatenate([edge_index[0], pad_ids]).reshape(NW, GROUPS, BATCH)
    dstp = jnp.concatenate([edge_index[1], pad_ids]).reshape(NW, GROUPS, BATCH)

    zeros16 = jnp.zeros((N_PAD, 16), f32)
    ones16 = jnp.ones((BATCH, 16), f32)
    zeros128 = jnp.zeros((N_PAD, 128), f32)
    zeros64 = jnp.zeros((N_PAD, 64), f32)
    zeros32 = jnp.zeros((N_PAD, 32), f32)

    degp = _sc_deg()(dstp, ones16, zeros16)
    dinv = _dinv_from_deg(degp)

    prop128 = _sc_prop(128)
    prop64 = _sc_prop(64)
    prop32 = _sc_prop(32)

    def mixhop3(h, Wcat, b0, b1, b2):
        z0, a = _dense_in(h, Wcat, b0, dinv, 64)
        p = prop128(a, zeros128, srcp, dstp)
        out1, g2 = _combine_mid(p, dinv, b1, 64)
        q = prop64(g2, zeros64, srcp, dstp)
        return _combine_out(z0, out1, q, dinv, b2)

    W1 = jnp.concatenate([conv1_W[0], conv1_W[1], conv1_W[2]], axis=1)
    h = mixhop3(xp, W1, conv1_b[0][None], conv1_b[1][None], conv1_b[2][None])

    for i in range(2):
        Wm = jnp.concatenate([block_W[i, 0], block_W[i, 1], block_W[i, 2]],
                             axis=1)
        h = mixhop3(h, Wm, block_b[i, 0][None], block_b[i, 1][None],
                    block_b[i, 2][None])

    W3 = jnp.concatenate([conv3_W[0], conv3_W[1]], axis=1)
    z0, a1 = _dense_in(h, W3, conv3_b[0][None], dinv, 32)
    q = prop32(a1, zeros32, srcp, dstp)
    out = _final_out(z0, q, dinv, conv3_b[1][None])
    return out[:N_NODES]


# speedup vs baseline: 16.7322x; 16.7322x over previous
import functools

import jax
import jax.numpy as jnp
from jax import lax
from jax.experimental import pallas as pl
from jax.experimental.pallas import tpu as pltpu
from jax.experimental.pallas import tpu_sc as plsc

N_NODES = 10000
N_PAD = 10240
N_EDGES = 320000
NCORES = 2
NSUB = 16
NW = NCORES * NSUB
BATCH = 128
GROUPS = 79
E_PAD = NW * GROUPS * BATCH
ROWS_PER_TILE = N_PAD // NSUB
PAD_SPREAD = 240


def _sc_prop(F):
    mesh = plsc.VectorSubcoreMesh(core_axis_name="c", subcore_axis_name="s")

    @functools.partial(
        pl.kernel,
        out_type=jax.ShapeDtypeStruct((NCORES, N_PAD, F), jnp.float32),
        mesh=mesh,
        scratch_types=[
            pltpu.VMEM_SHARED((N_PAD, F), jnp.float32),
            pltpu.VMEM((GROUPS, BATCH), jnp.int32),
            pltpu.VMEM((GROUPS, BATCH), jnp.int32),
            pltpu.VMEM((BATCH, F), jnp.float32),
            pltpu.SemaphoreType.DMA,
        ],
    )
    def prop(z_hbm, zeros_hbm, src_hbm, dst_hbm, out_hbm,
             acc, src_v, dst_v, rows, sem):
        c = lax.axis_index("c")
        s = lax.axis_index("s")
        w = c * NSUB + s
        pltpu.sync_copy(src_hbm.at[w], src_v)
        pltpu.sync_copy(dst_hbm.at[w], dst_v)
        r0 = s * ROWS_PER_TILE

        @pl.when(c == 0)
        def _():
            pltpu.sync_copy(z_hbm.at[pl.ds(r0, ROWS_PER_TILE)],
                            acc.at[pl.ds(r0, ROWS_PER_TILE)])

        @pl.when(c != 0)
        def _():
            pltpu.sync_copy(zeros_hbm.at[pl.ds(r0, ROWS_PER_TILE)],
                            acc.at[pl.ds(r0, ROWS_PER_TILE)])

        plsc.subcore_barrier()

        def body(g, carry):
            pltpu.async_copy(z_hbm.at[src_v.at[g]], rows, sem).wait()
            pltpu.sync_copy(rows, acc.at[dst_v.at[g]], add=True)
            return carry

        lax.fori_loop(0, GROUPS, body, 0)
        plsc.subcore_barrier()
        pltpu.sync_copy(acc.at[pl.ds(r0, ROWS_PER_TILE)],
                        out_hbm.at[c, pl.ds(r0, ROWS_PER_TILE)])

    return prop


def _sc_deg():
    mesh = plsc.VectorSubcoreMesh(core_axis_name="c", subcore_axis_name="s")

    @functools.partial(
        pl.kernel,
        out_type=jax.ShapeDtypeStruct((NCORES, N_PAD, 128), jnp.float32),
        mesh=mesh,
        scratch_types=[
            pltpu.VMEM_SHARED((N_PAD, 128), jnp.float32),
            pltpu.VMEM((GROUPS, BATCH), jnp.int32),
            pltpu.VMEM((BATCH, 128), jnp.float32),
        ],
    )
    def deg(dst_hbm, ones_hbm, zeros_hbm, out_hbm, acc, dst_v, ones_v):
        c = lax.axis_index("c")
        s = lax.axis_index("s")
        w = c * NSUB + s
        pltpu.sync_copy(dst_hbm.at[w], dst_v)
        pltpu.sync_copy(ones_hbm, ones_v)
        r0 = s * ROWS_PER_TILE
        pltpu.sync_copy(zeros_hbm.at[pl.ds(r0, ROWS_PER_TILE)],
                        acc.at[pl.ds(r0, ROWS_PER_TILE)])
        plsc.subcore_barrier()

        def body(g, carry):
            pltpu.sync_copy(ones_v, acc.at[dst_v.at[g]], add=True)
            return carry

        lax.fori_loop(0, GROUPS, body, 0)
        plsc.subcore_barrier()
        pltpu.sync_copy(acc.at[pl.ds(r0, ROWS_PER_TILE)],
                        out_hbm.at[c, pl.ds(r0, ROWS_PER_TILE)])

    return deg


_BM = 1024


def _dinv_from_deg(degp):
    def body(p_ref, o_ref):
        deg = p_ref[0, :, 0:1] + p_ref[1, :, 0:1] + 1.0
        safe = jnp.maximum(deg, 1e-12)
        o_ref[...] = jnp.where(deg > 0, lax.rsqrt(safe), 0.0)

    return pl.pallas_call(
        body,
        grid=(N_PAD // _BM,),
        in_specs=[pl.BlockSpec((2, _BM, 128), lambda i: (0, i, 0))],
        out_specs=pl.BlockSpec((_BM, 1), lambda i: (i, 0)),
        out_shape=jax.ShapeDtypeStruct((N_PAD, 1), jnp.float32),
    )(degp)


def _dense_in(h, W, b0, dinv, d0, apad):
    K = h.shape[1]
    dtot = W.shape[1]
    da = dtot - d0

    def body(h_ref, w_ref, b_ref, dv_ref, z0_ref, a_ref):
        prod = jnp.dot(h_ref[...], w_ref[...],
                       preferred_element_type=jnp.float32)
        z0_ref[...] = prod[:, :d0] + b_ref[...]
        av = prod[:, d0:] * dv_ref[...]
        if apad > da:
            av = jnp.concatenate(
                [av, jnp.zeros((av.shape[0], apad - da), jnp.float32)], axis=1)
        a_ref[...] = av

    return pl.pallas_call(
        body,
        grid=(N_PAD // _BM,),
        in_specs=[
            pl.BlockSpec((_BM, K), lambda i: (i, 0)),
            pl.BlockSpec((K, dtot), lambda i: (0, 0)),
            pl.BlockSpec((1, d0), lambda i: (0, 0)),
            pl.BlockSpec((_BM, 1), lambda i: (i, 0)),
        ],
        out_specs=[
            pl.BlockSpec((_BM, d0), lambda i: (i, 0)),
            pl.BlockSpec((_BM, apad), lambda i: (i, 0)),
        ],
        out_shape=[
            jax.ShapeDtypeStruct((N_PAD, d0), jnp.float32),
            jax.ShapeDtypeStruct((N_PAD, apad), jnp.float32),
        ],
    )(h, W, b0, dinv)


def _combine_mid(p, dinv, b1, dh):
    F = p.shape[2]
    da = F - dh

    def body(p_ref, dv_ref, b_ref, o1_ref, g2_ref):
        sm = p_ref[0] + p_ref[1]
        dv = dv_ref[...]
        o1_ref[...] = sm[:, :dh] * dv + b_ref[...]
        gv = sm[:, dh:] * (dv * dv)
        g2_ref[...] = jnp.concatenate(
            [gv, jnp.zeros((gv.shape[0], 128 - da), jnp.float32)], axis=1)

    return pl.pallas_call(
        body,
        grid=(N_PAD // _BM,),
        in_specs=[
            pl.BlockSpec((2, _BM, F), lambda i: (0, i, 0)),
            pl.BlockSpec((_BM, 1), lambda i: (i, 0)),
            pl.BlockSpec((1, dh), lambda i: (0, 0)),
        ],
        out_specs=[
            pl.BlockSpec((_BM, dh), lambda i: (i, 0)),
            pl.BlockSpec((_BM, 128), lambda i: (i, 0)),
        ],
        out_shape=[
            jax.ShapeDtypeStruct((N_PAD, dh), jnp.float32),
            jax.ShapeDtypeStruct((N_PAD, 128), jnp.float32),
        ],
    )(p, dinv, b1)


def _combine_out(z0, out1, q, dinv, b2, d2):
    d0 = z0.shape[1]
    d1 = out1.shape[1]
    Fq = q.shape[2]

    def body(z0_ref, o1_ref, q_ref, dv_ref, b_ref, h_ref):
        o2 = (q_ref[0, :, :d2] + q_ref[1, :, :d2]) * dv_ref[...] + b_ref[...]
        h_ref[...] = jnp.concatenate([z0_ref[...], o1_ref[...], o2], axis=1)

    return pl.pallas_call(
        body,
        grid=(N_PAD // _BM,),
        in_specs=[
            pl.BlockSpec((_BM, d0), lambda i: (i, 0)),
            pl.BlockSpec((_BM, d1), lambda i: (i, 0)),
            pl.BlockSpec((2, _BM, Fq), lambda i: (0, i, 0)),
            pl.BlockSpec((_BM, 1), lambda i: (i, 0)),
            pl.BlockSpec((1, d2), lambda i: (0, 0)),
        ],
        out_specs=pl.BlockSpec((_BM, d0 + d1 + d2), lambda i: (i, 0)),
        out_shape=jax.ShapeDtypeStruct((N_PAD, d0 + d1 + d2), jnp.float32),
    )(z0, out1, q, dinv, b2)


def _final_out(z0, q, dinv, b1, d1):
    d0 = z0.shape[1]
    Fq = q.shape[2]

    def body(z0_ref, q_ref, dv_ref, b_ref, h_ref):
        o1 = (q_ref[0, :, :d1] + q_ref[1, :, :d1]) * dv_ref[...] + b_ref[...]
        h_ref[...] = jnp.concatenate([z0_ref[...], o1], axis=1)

    return pl.pallas_call(
        body,
        grid=(N_PAD // _BM,),
        in_specs=[
            pl.BlockSpec((_BM, d0), lambda i: (i, 0)),
            pl.BlockSpec((2, _BM, Fq), lambda i: (0, i, 0)),
            pl.BlockSpec((_BM, 1), lambda i: (i, 0)),
            pl.BlockSpec((1, d1), lambda i: (0, 0)),
        ],
        out_specs=pl.BlockSpec((_BM, d0 + d1), lambda i: (i, 0)),
        out_shape=jax.ShapeDtypeStruct((N_PAD, d0 + d1), jnp.float32),
    )(z0, q, dinv, b1)


def kernel(x, edge_index, conv1_W, conv1_b, block_W, block_b, conv3_W, conv3_b):
    f32 = jnp.float32

    xp = jnp.pad(x, ((0, N_PAD - N_NODES), (0, 0)))
    npad = E_PAD - N_EDGES
    pad_ids = (jnp.arange(npad, dtype=jnp.int32) % PAD_SPREAD) + N_NODES
    srcp = jnp.concatenate([edge_index[0], pad_ids]).reshape(NW, GROUPS, BATCH)
    dstp = jnp.concatenate([edge_index[1], pad_ids]).reshape(NW, GROUPS, BATCH)

    ones128 = jnp.ones((BATCH, 128), f32)
    zeros128 = jnp.zeros((N_PAD, 128), f32)

    degp = _sc_deg()(dstp, ones128, zeros128)
    dinv = _dinv_from_deg(degp)

    prop128 = _sc_prop(128)

    def mixhop3(h, Wcat, b0, b1, b2):
        z0, a = _dense_in(h, Wcat, b0, dinv, 64, 128)
        p = prop128(a, zeros128, srcp, dstp)
        out1, g2 = _combine_mid(p, dinv, b1, 64)
        q = prop128(g2, zeros128, srcp, dstp)
        return _combine_out(z0, out1, q, dinv, b2, 64)

    W1 = jnp.concatenate([conv1_W[0], conv1_W[1], conv1_W[2]], axis=1)
    h = mixhop3(xp, W1, conv1_b[0][None], conv1_b[1][None], conv1_b[2][None])

    for i in range(2):
        Wm = jnp.concatenate([block_W[i, 0], block_W[i, 1], block_W[i, 2]],
                             axis=1)
        h = mixhop3(h, Wm, block_b[i, 0][None], block_b[i, 1][None],
                    block_b[i, 2][None])

    W3 = jnp.concatenate([conv3_W[0], conv3_W[1]], axis=1)
    z0, a1 = _dense_in(h, W3, conv3_b[0][None], dinv, 32, 128)
    q = prop128(a1, zeros128, srcp, dstp)
    out = _final_out(z0, q, dinv, conv3_b[1][None], 32)
    return out[:N_NODES]
